# Initial kernel scaffold; baseline (speedup 1.0000x reference)
import jax
import jax.numpy as jnp
from jax.experimental import pallas as pl


def kernel(input, target):
    raise NotImplementedError("write your pallas kernel here")


# speedup vs baseline: 12.7521x; 12.7521x over previous
import functools

import jax
import jax.numpy as jnp
from jax import lax
from jax.experimental import pallas as pl
from jax.experimental.pallas import tpu as pltpu

_BS = 1000
_CB = 1024


def _hd95_body(px_ref, py_ref, pv_ref, tx_ref, ty_ref, tv_ref,
               pxr_ref, pyr_ref, pvr_ref, txr_ref, tyr_ref, tvr_ref,
               out_ref, rows_ref, cols_ref, sm_ref,
               *, C, maxd2, iters, nbatch):
    b = pl.program_id(0)

    @pl.when(b == 0)
    def _init():
        sm_ref[0] = jnp.float32(0.0)
        sm_ref[1] = jnp.float32(0.0)

    n_pred = jnp.sum(pv_ref[0]).astype(jnp.int32)
    n_true = jnp.sum(tv_ref[0]).astype(jnp.int32)
    n_pc = (n_pred + _BS - 1) // _BS
    n_tc = (n_true + _BS - 1) // _BS
    npairs = n_pc * n_tc

    inf = jnp.float32(jnp.inf)

    def _strip_mins(axT, ayT, bx_r, by_r, bok_r):
        mins = []
        for s in range(8):
            qx = axT[:, s:s + 1]
            qy = ayT[:, s:s + 1]
            dx = qx - bx_r
            dy = qy - by_r
            d2 = dx * dx + dy * dy
            m = jnp.min(jnp.where(bok_r > 0, d2, inf), axis=1, keepdims=True)
            mins.append(m)
        return jnp.transpose(jnp.concatenate(mins, axis=1))

    def pair_body(k, carry):
        i = k // n_tc
        j = k - i * n_tc
        pxT = jnp.transpose(px_ref[0, i])
        pyT = jnp.transpose(py_ref[0, i])
        rows_ref[k] = _strip_mins(pxT, pyT, txr_ref[0, j], tyr_ref[0, j],
                                  tvr_ref[0, j])
        txT = jnp.transpose(tx_ref[0, j])
        tyT = jnp.transpose(ty_ref[0, j])
        cols_ref[k] = _strip_mins(txT, tyT, pxr_ref[0, i], pyr_ref[0, i],
                                  pvr_ref[0, i])
        return carry

    lax.fori_loop(0, npairs, pair_body, 0)

    n = n_pred * n_tc + n_true * n_pc
    nf = n.astype(jnp.float32)
    q = jnp.float32(0.95) * (nf - jnp.float32(1))
    low = jnp.floor(q)
    high = jnp.ceil(q)
    high_w = q - low
    low_w = jnp.float32(1) - high_w
    low_i = jnp.maximum(jnp.float32(0), jnp.minimum(low, nf - 1)).astype(jnp.int32)
    high_i = jnp.maximum(jnp.float32(0), jnp.minimum(high, nf - 1)).astype(jnp.int32)
    k1f = (low_i + 1).astype(jnp.float32)
    k2f = (high_i + 1).astype(jnp.float32)

    def count2(t1, t2):
        t1f = t1.astype(jnp.float32)
        t2f = t2.astype(jnp.float32)

        def cbody(k, accs):
            a1, a2 = accs
            i = k // n_tc
            j = k - i * n_tc
            rm = rows_ref[k]
            cm = cols_ref[k]
            pok = pv_ref[0, i] > 0
            tok = tv_ref[0, j] > 0
            one = jnp.float32(1.0)
            zero = jnp.float32(0.0)
            a1 = a1 + jnp.where(pok & (rm <= t1f), one, zero) \
                    + jnp.where(tok & (cm <= t1f), one, zero)
            a2 = a2 + jnp.where(pok & (rm <= t2f), one, zero) \
                    + jnp.where(tok & (cm <= t2f), one, zero)
            return a1, a2

        z = jnp.zeros((8, 128), jnp.float32)
        a1, a2 = lax.fori_loop(0, npairs, cbody, (z, z))
        return jnp.sum(a1), jnp.sum(a2)

    def bs_body(_, st):
        lo1, hi1, lo2, hi2 = st
        mid1 = (lo1 + hi1) // 2
        mid2 = (lo2 + hi2) // 2
        c1, c2 = count2(mid1, mid2)
        ok1 = c1 >= k1f
        ok2 = c2 >= k2f
        hi1 = jnp.where(ok1, mid1, hi1)
        lo1 = jnp.where(ok1, lo1, mid1 + 1)
        hi2 = jnp.where(ok2, mid2, hi2)
        lo2 = jnp.where(ok2, lo2, mid2 + 1)
        return lo1, hi1, lo2, hi2

    zero_i = jnp.int32(0)
    max_i = jnp.int32(maxd2)
    lo1, _, lo2, _ = lax.fori_loop(0, iters, bs_body,
                                   (zero_i, max_i, zero_i, max_i))

    hd = (jnp.sqrt(lo1.astype(jnp.float32)) * low_w
          + jnp.sqrt(lo2.astype(jnp.float32)) * high_w)
    valid = (n_pred > 0) & (n_true > 0)
    sm_ref[0] = sm_ref[0] + jnp.where(valid, hd, jnp.float32(0.0))
    sm_ref[1] = sm_ref[1] + jnp.where(valid, jnp.float32(1.0), jnp.float32(0.0))

    total = sm_ref[0]
    cnt = sm_ref[1]
    out_ref[...] = jnp.where(cnt > 0, total / cnt, inf).reshape(1, 1)


def _compact(mask, cx, cy, C, pad):
    order = jnp.argsort(~mask)
    xs = cx[order]
    ys = cy[order]
    vv = mask[order].astype(jnp.float32)
    z = jnp.zeros((pad,), jnp.float32)
    xs = jnp.concatenate([xs, z]).reshape(C, _BS)
    ys = jnp.concatenate([ys, z]).reshape(C, _BS)
    vv = jnp.concatenate([vv, z]).reshape(C, _BS)
    padc = ((0, 0), (0, _CB - _BS))
    xs = jnp.pad(xs, padc)
    ys = jnp.pad(ys, padc)
    vv = jnp.pad(vv, padc)
    tile = lambda a: a.reshape(C, 8, 128)
    row = lambda a: a.reshape(C, 1, _CB)
    return tile(xs), tile(ys), tile(vv), row(xs), row(ys), row(vv)


def kernel(input, target):
    B, H, W = input.shape
    N = H * W
    C = -(-N // _BS)
    pad = C * _BS - N
    maxd2 = (H - 1) ** 2 + (W - 1) ** 2
    iters = max(1, (maxd2 + 1).bit_length())

    pm = (input > 0.5).reshape(B, N)
    tm = (target > 0.5).reshape(B, N)

    rr, cc = jnp.meshgrid(jnp.arange(H), jnp.arange(W), indexing="ij")
    cx = rr.reshape(N).astype(jnp.float32)
    cy = cc.reshape(N).astype(jnp.float32)

    comp = jax.vmap(functools.partial(_compact, cx=cx, cy=cy, C=C, pad=pad))
    px, py, pv, pxr, pyr, pvr = comp(pm)
    tx, ty, tv, txr, tyr, tvr = comp(tm)

    tspec = pl.BlockSpec((1, C, 8, 128), lambda b: (b, 0, 0, 0))
    rspec = pl.BlockSpec((1, C, 1, _CB), lambda b: (b, 0, 0, 0))
    body = functools.partial(_hd95_body, C=C, maxd2=maxd2, iters=iters,
                             nbatch=B)
    out = pl.pallas_call(
        body,
        grid=(B,),
        in_specs=[tspec] * 6 + [rspec] * 6,
        out_specs=pl.BlockSpec((1, 1), lambda b: (0, 0)),
        out_shape=jax.ShapeDtypeStruct((1, 1), jnp.float32),
        scratch_shapes=[
            pltpu.VMEM((C * C, 8, 128), jnp.float32),
            pltpu.VMEM((C * C, 8, 128), jnp.float32),
            pltpu.SMEM((2,), jnp.float32),
        ],
    )(px, py, pv, tx, ty, tv, pxr, pyr, pvr, txr, tyr, tvr)
    return out.reshape(())
